# Initial kernel scaffold; baseline (speedup 1.0000x reference)
import functools

import jax
import jax.numpy as jnp
from jax import lax
from jax.experimental import pallas as pl
from jax.experimental.pallas import tpu as pltpu
from jax.experimental.pallas import tpu_sc as plsc

N = 10000
E = 320000
F = 128
C = 40

NC = 2
NS = 16
NW = NC * NS
EPW = E // NW
B = 80
CH = EPW // B
RPS = N // NS
LANES = 16


def _spmm_sc(src3, dst3, wflat, feats):
    mesh = plsc.VectorSubcoreMesh(core_axis_name="c", subcore_axis_name="s")

    @functools.partial(
        pl.kernel,
        mesh=mesh,
        out_type=jax.ShapeDtypeStruct((NC, N, F), jnp.float32),
        scratch_types=[
            pltpu.VMEM((CH, B), jnp.int32),
            pltpu.VMEM((CH, B), jnp.int32),
            pltpu.VMEM((EPW,), jnp.float32),
            pltpu.VMEM((B, F), jnp.float32),
            pltpu.VMEM_SHARED((N, F), jnp.float32),
        ],
    )
    def k(src_hbm, dst_hbm, w_hbm, feats_hbm, out_hbm, src_v, dst_v, w_v, buf, acc):
        c = lax.axis_index("c")
        s = lax.axis_index("s")

        pltpu.sync_copy(src_hbm.at[c, s], src_v)
        pltpu.sync_copy(dst_hbm.at[c, s], dst_v)
        pltpu.sync_copy(w_hbm.at[c, s], w_v)

        zero = jnp.zeros((LANES,), jnp.float32)
        for i in range(B):
            for cc in range(F // LANES):
                buf[i, pl.ds(cc * LANES, LANES)] = zero
        row0 = s * RPS
        for t in range(RPS // B):
            pltpu.sync_copy(buf, acc.at[pl.ds(row0 + t * B, B)])
        rem = RPS % B
        if rem:
            pltpu.sync_copy(buf.at[pl.ds(0, rem)],
                            acc.at[pl.ds(row0 + (RPS // B) * B, rem)])
        plsc.subcore_barrier()

        @pl.loop(0, CH)
        def _(kk):
            pltpu.sync_copy(feats_hbm.at[src_v.at[kk]], buf)
            base = kk * B
            for i in range(B):
                wv = plsc.load_gather(
                    w_v, [jnp.full((LANES,), base + i, dtype=jnp.int32)])
                for cc in range(F // LANES):
                    sl = (i, pl.ds(cc * LANES, LANES))
                    buf[sl] = buf[sl] * wv
            pltpu.sync_copy(buf, acc.at[dst_v.at[kk]], add=True)

        plsc.subcore_barrier()
        pltpu.sync_copy(acc.at[pl.ds(row0, RPS)], out_hbm.at[c, pl.ds(row0, RPS)])

    return k(src3, dst3, wflat, feats)


_ROWS = 2000


def _linear_tc(a, W, b):
    def body(a_ref, w_ref, b_ref, o_ref):
        o_ref[...] = (
            jnp.dot(a_ref[...], w_ref[...], preferred_element_type=jnp.float32)
            + b_ref[...]
        )

    return pl.pallas_call(
        body,
        grid=(N // _ROWS,),
        in_specs=[
            pl.BlockSpec((_ROWS, F), lambda i: (i, 0)),
            pl.BlockSpec((F, F), lambda i: (0, 0)),
            pl.BlockSpec((1, F), lambda i: (0, 0)),
        ],
        out_specs=pl.BlockSpec((_ROWS, F), lambda i: (i, 0)),
        out_shape=jax.ShapeDtypeStruct((N, F), jnp.float32),
    )(a, W, b.reshape(1, F))


def _fused_linear_tc(p, W, b):
    def body(pa_ref, pb_ref, w_ref, b_ref, o_ref):
        h = jnp.maximum(pa_ref[0] + pb_ref[0], 0.0)
        o_ref[...] = (
            jnp.dot(h, w_ref[...], preferred_element_type=jnp.float32)
            + b_ref[...]
        )

    return pl.pallas_call(
        body,
        grid=(N // _ROWS,),
        in_specs=[
            pl.BlockSpec((1, _ROWS, F), lambda i: (0, i, 0)),
            pl.BlockSpec((1, _ROWS, F), lambda i: (1, i, 0)),
            pl.BlockSpec((F, F), lambda i: (0, 0)),
            pl.BlockSpec((1, F), lambda i: (0, 0)),
        ],
        out_specs=pl.BlockSpec((_ROWS, F), lambda i: (i, 0)),
        out_shape=jax.ShapeDtypeStruct((N, F), jnp.float32),
    )(p, p, W, b.reshape(1, F))


def _final_tc(p, Wfc, bfc):
    def body(pa_ref, pb_ref, w_ref, b_ref, o_ref):
        z = jnp.maximum(pa_ref[0] + pb_ref[0], 0.0)
        logits = (
            jnp.dot(z, w_ref[...], preferred_element_type=jnp.float32)
            + b_ref[...]
        )
        m = jnp.max(logits, axis=1, keepdims=True)
        e = jnp.exp(logits - m)
        lse = jnp.log(jnp.sum(e, axis=1, keepdims=True))
        o_ref[...] = logits - m - lse

    return pl.pallas_call(
        body,
        grid=(N // _ROWS,),
        in_specs=[
            pl.BlockSpec((1, _ROWS, F), lambda i: (0, i, 0)),
            pl.BlockSpec((1, _ROWS, F), lambda i: (1, i, 0)),
            pl.BlockSpec((F, C), lambda i: (0, 0)),
            pl.BlockSpec((1, C), lambda i: (0, 0)),
        ],
        out_specs=pl.BlockSpec((_ROWS, C), lambda i: (i, 0)),
        out_shape=jax.ShapeDtypeStruct((N, C), jnp.float32),
    )(p, p, Wfc, bfc.reshape(1, C))


def kernel(x, edge_index, edge_weight, W1, b1, W2, b2, Wfc, bfc):
    src3 = edge_index[0].astype(jnp.int32).reshape(NC, NS, CH, B)
    dst3 = edge_index[1].astype(jnp.int32).reshape(NC, NS, CH, B)
    wflat = edge_weight.astype(jnp.float32).reshape(NC, NS, EPW)

    s1 = _linear_tc(x, W1, b1)
    p1 = _spmm_sc(src3, dst3, wflat, s1)
    s2 = _fused_linear_tc(p1, W2, b2)
    p2 = _spmm_sc(src3, dst3, wflat, s2)
    return _final_tc(p2, Wfc, bfc)


# speedup vs baseline: 5.1054x; 5.1054x over previous
import dataclasses
import functools

import jax
import jax.numpy as jnp
from jax import lax
from jax.experimental import pallas as pl
from jax.experimental.pallas import tpu as pltpu
from jax.experimental.pallas import tpu_sc as plsc

N = 10000
E = 320000
F = 128
C = 40

NC = 2
NS = 16
NW = NC * NS
EPW = E // NW
B = 80
CH = EPW // B
SUP = 25
NSUP = CH // SUP
RPS = 624
TAIL = N - RPS * NS
LANES = 16


def _spmm_sc(src3, dst3, wflat, feats):
    mesh = plsc.VectorSubcoreMesh(core_axis_name="c", subcore_axis_name="s")
    cp = pltpu.CompilerParams()
    if "needs_layout_passes" in pltpu.CompilerParams.__dataclass_fields__:
        cp = dataclasses.replace(cp, needs_layout_passes=False)

    @functools.partial(
        pl.kernel,
        mesh=mesh,
        compiler_params=cp,
        out_type=jax.ShapeDtypeStruct((NC, N, F), jnp.float32),
        scratch_types=[
            pltpu.VMEM((SUP, B), jnp.int32),
            pltpu.VMEM((SUP, B), jnp.int32),
            pltpu.VMEM((SUP * B,), jnp.float32),
            pltpu.VMEM((B, F), jnp.float32),
            pltpu.VMEM_SHARED((N, F), jnp.float32),
        ],
    )
    def k(src_hbm, dst_hbm, w_hbm, feats_hbm, out_hbm, src_v, dst_v, w_v, buf, acc):
        c = lax.axis_index("c")
        s = lax.axis_index("s")
        wid = c * NS + s

        zero = jnp.zeros((LANES,), jnp.float32)
        for i in range(B):
            for cc in range(F // LANES):
                buf[i, pl.ds(cc * LANES, LANES)] = zero
        row0 = s * RPS
        for t in range(RPS // B):
            pltpu.sync_copy(buf, acc.at[pl.ds(row0 + t * B, B)])
        rem = RPS % B
        if rem:
            pltpu.sync_copy(buf.at[pl.ds(0, rem)],
                            acc.at[pl.ds(row0 + (RPS // B) * B, rem)])

        @pl.when(s == NS - 1)
        def _():
            pltpu.sync_copy(buf.at[pl.ds(0, TAIL)], acc.at[pl.ds(RPS * NS, TAIL)])

        plsc.subcore_barrier()

        @pl.loop(0, NSUP)
        def _(g):
            pltpu.sync_copy(src_hbm.at[wid * NSUP + g], src_v)
            pltpu.sync_copy(dst_hbm.at[wid * NSUP + g], dst_v)
            pltpu.sync_copy(w_hbm.at[wid * NSUP + g], w_v)

            @pl.loop(0, SUP)
            def _(kk):
                pltpu.sync_copy(feats_hbm.at[src_v.at[kk]], buf)
                base = kk * B
                for i in range(B):
                    wv = plsc.load_gather(
                        w_v, [jnp.full((LANES,), base + i, dtype=jnp.int32)])
                    for cc in range(F // LANES):
                        sl = (i, pl.ds(cc * LANES, LANES))
                        buf[sl] = buf[sl] * wv
                pltpu.sync_copy(buf, acc.at[dst_v.at[kk]], add=True)

        plsc.subcore_barrier()
        pltpu.sync_copy(acc.at[pl.ds(row0, RPS)], out_hbm.at[c, pl.ds(row0, RPS)])

        @pl.when(s == NS - 1)
        def _():
            pltpu.sync_copy(acc.at[pl.ds(RPS * NS, TAIL)],
                            out_hbm.at[c, pl.ds(RPS * NS, TAIL)])

    return k(src3, dst3, wflat, feats)


_ROWS = 2000


def _linear_tc(a, W, b):
    def body(a_ref, w_ref, b_ref, o_ref):
        o_ref[...] = (
            jnp.dot(a_ref[...], w_ref[...], preferred_element_type=jnp.float32)
            + b_ref[...]
        )

    return pl.pallas_call(
        body,
        grid=(N // _ROWS,),
        in_specs=[
            pl.BlockSpec((_ROWS, F), lambda i: (i, 0)),
            pl.BlockSpec((F, F), lambda i: (0, 0)),
            pl.BlockSpec((1, F), lambda i: (0, 0)),
        ],
        out_specs=pl.BlockSpec((_ROWS, F), lambda i: (i, 0)),
        out_shape=jax.ShapeDtypeStruct((N, F), jnp.float32),
    )(a, W, b.reshape(1, F))


def _fused_linear_tc(p, W, b):
    def body(pa_ref, pb_ref, w_ref, b_ref, o_ref):
        h = jnp.maximum(pa_ref[0] + pb_ref[0], 0.0)
        o_ref[...] = (
            jnp.dot(h, w_ref[...], preferred_element_type=jnp.float32)
            + b_ref[...]
        )

    return pl.pallas_call(
        body,
        grid=(N // _ROWS,),
        in_specs=[
            pl.BlockSpec((1, _ROWS, F), lambda i: (0, i, 0)),
            pl.BlockSpec((1, _ROWS, F), lambda i: (1, i, 0)),
            pl.BlockSpec((F, F), lambda i: (0, 0)),
            pl.BlockSpec((1, F), lambda i: (0, 0)),
        ],
        out_specs=pl.BlockSpec((_ROWS, F), lambda i: (i, 0)),
        out_shape=jax.ShapeDtypeStruct((N, F), jnp.float32),
    )(p, p, W, b.reshape(1, F))


def _final_tc(p, Wfc, bfc):
    def body(pa_ref, pb_ref, w_ref, b_ref, o_ref):
        z = jnp.maximum(pa_ref[0] + pb_ref[0], 0.0)
        logits = (
            jnp.dot(z, w_ref[...], preferred_element_type=jnp.float32)
            + b_ref[...]
        )
        m = jnp.max(logits, axis=1, keepdims=True)
        e = jnp.exp(logits - m)
        lse = jnp.log(jnp.sum(e, axis=1, keepdims=True))
        o_ref[...] = logits - m - lse

    return pl.pallas_call(
        body,
        grid=(N // _ROWS,),
        in_specs=[
            pl.BlockSpec((1, _ROWS, F), lambda i: (0, i, 0)),
            pl.BlockSpec((1, _ROWS, F), lambda i: (1, i, 0)),
            pl.BlockSpec((F, C), lambda i: (0, 0)),
            pl.BlockSpec((1, C), lambda i: (0, 0)),
        ],
        out_specs=pl.BlockSpec((_ROWS, C), lambda i: (i, 0)),
        out_shape=jax.ShapeDtypeStruct((N, C), jnp.float32),
    )(p, p, Wfc, bfc.reshape(1, C))


def kernel(x, edge_index, edge_weight, W1, b1, W2, b2, Wfc, bfc):
    src3 = edge_index[0].astype(jnp.int32).reshape(NW * NSUP, SUP, B)
    dst3 = edge_index[1].astype(jnp.int32).reshape(NW * NSUP, SUP, B)
    wflat = edge_weight.astype(jnp.float32).reshape(NW * NSUP, SUP * B)

    s1 = _linear_tc(x, W1, b1)
    p1 = _spmm_sc(src3, dst3, wflat, s1)
    s2 = _fused_linear_tc(p1, W2, b2)
    p2 = _spmm_sc(src3, dst3, wflat, s2)
    return _final_tc(p2, Wfc, bfc)
